# Initial kernel scaffold; baseline (speedup 1.0000x reference)
import jax
import jax.numpy as jnp
from jax.experimental import pallas as pl


def kernel(position_ids, inv_freq):
    raise NotImplementedError("write your pallas kernel here")


# speedup vs baseline: 1.8685x; 1.8685x over previous
import jax
import jax.numpy as jnp
from jax.experimental import pallas as pl

_BLK = 2048
_HALF = 64
_DIM = 128


def _rope_kernel(pos_ref, freq_ref, cos_ref, sin_ref):
    pos = pos_ref[:, :].astype(jnp.float32)
    angles = pos * freq_ref[0:1, :]
    c = jnp.cos(angles)
    s = jnp.sin(angles)
    cos_ref[:, :] = jnp.concatenate([c, c], axis=1)
    sin_ref[:, :] = jnp.concatenate([s, s], axis=1)


def kernel(position_ids, inv_freq):
    b, s = position_ids.shape
    total = b * s
    pos = position_ids.reshape(total, 1)
    freq = inv_freq.reshape(1, _HALF)
    grid = (total // _BLK,)
    cos, sin = pl.pallas_call(
        _rope_kernel,
        grid=grid,
        in_specs=[
            pl.BlockSpec((_BLK, 1), lambda i: (i, 0)),
            pl.BlockSpec((1, _HALF), lambda i: (0, 0)),
        ],
        out_specs=[
            pl.BlockSpec((_BLK, _DIM), lambda i: (i, 0)),
            pl.BlockSpec((_BLK, _DIM), lambda i: (i, 0)),
        ],
        out_shape=[
            jax.ShapeDtypeStruct((total, _DIM), jnp.float32),
            jax.ShapeDtypeStruct((total, _DIM), jnp.float32),
        ],
    )(pos, freq)
    return (cos.reshape(b, s, 1, _DIM), sin.reshape(b, s, 1, _DIM))


# speedup vs baseline: 3.7269x; 1.9946x over previous
import jax
import jax.numpy as jnp
from jax import lax
from jax.experimental import pallas as pl

_BLK = 2048
_DIM = 128

_C1 = -1.2337005501361697
_C2 = 0.25366950790104696
_C3 = -0.020863480763352957
_S0 = 1.5707963267948966
_S1 = -0.6459640975062462
_S2 = 0.07969262624616703
_S3 = -0.0046817541353186846


def _rope_kernel(pos_ref, freq_ref, cos_ref, sin_ref):
    pos = pos_ref[:, :].astype(jnp.float32)
    x = pos * freq_ref[0:1, :]
    k = jnp.round(x)
    t = x - k
    q = k.astype(jnp.int32)
    t2 = t * t
    cp = 1.0 + t2 * (_C1 + t2 * (_C2 + t2 * _C3))
    sp = t * (_S0 + t2 * (_S1 + t2 * (_S2 + t2 * _S3)))
    swap = (q & 1) != 0
    c0 = jnp.where(swap, sp, cp)
    s0 = jnp.where(swap, cp, sp)
    sgn_c = ((q + 1) & 2) << 30
    sgn_s = (q & 2) << 30
    cos_ref[:, :] = lax.bitcast_convert_type(
        lax.bitcast_convert_type(c0, jnp.int32) ^ sgn_c, jnp.float32)
    sin_ref[:, :] = lax.bitcast_convert_type(
        lax.bitcast_convert_type(s0, jnp.int32) ^ sgn_s, jnp.float32)


def kernel(position_ids, inv_freq):
    b, s = position_ids.shape
    total = b * s
    pos = position_ids.reshape(total, 1)
    fq = (inv_freq * (2.0 / jnp.pi)).astype(jnp.float32)
    freq = jnp.concatenate([fq, fq]).reshape(1, _DIM)
    grid = (total // _BLK,)
    cos, sin = pl.pallas_call(
        _rope_kernel,
        grid=grid,
        in_specs=[
            pl.BlockSpec((_BLK, 1), lambda i: (i, 0)),
            pl.BlockSpec((1, _DIM), lambda i: (0, 0)),
        ],
        out_specs=[
            pl.BlockSpec((_BLK, _DIM), lambda i: (i, 0)),
            pl.BlockSpec((_BLK, _DIM), lambda i: (i, 0)),
        ],
        out_shape=[
            jax.ShapeDtypeStruct((total, _DIM), jnp.float32),
            jax.ShapeDtypeStruct((total, _DIM), jnp.float32),
        ],
    )(pos, freq)
    return (cos.reshape(b, s, 1, _DIM), sin.reshape(b, s, 1, _DIM))


# speedup vs baseline: 3.7449x; 1.0048x over previous
import jax
import jax.numpy as jnp
from jax import lax
from jax.experimental import pallas as pl
from jax.experimental.pallas import tpu as pltpu

_BLK = 2048
_DIM = 128

_C1 = -1.2337005501361697
_C2 = 0.25366950790104696
_C3 = -0.020863480763352957
_S0 = 1.5707963267948966
_S1 = -0.6459640975062462
_S2 = 0.07969262624616703
_S3 = -0.0046817541353186846


def _rope_kernel(pos_ref, freq_ref, cos_ref, sin_ref):
    pos = pos_ref[:, :].astype(jnp.float32)
    x = pos * freq_ref[0:1, :]
    k = jnp.round(x)
    t = x - k
    q = k.astype(jnp.int32)
    t2 = t * t
    cp = 1.0 + t2 * (_C1 + t2 * (_C2 + t2 * _C3))
    sp = t * (_S0 + t2 * (_S1 + t2 * (_S2 + t2 * _S3)))
    swap = (q & 1) != 0
    c0 = jnp.where(swap, sp, cp)
    s0 = jnp.where(swap, cp, sp)
    sgn_c = ((q + 1) & 2) << 30
    sgn_s = (q & 2) << 30
    cos_ref[:, :] = lax.bitcast_convert_type(
        lax.bitcast_convert_type(c0, jnp.int32) ^ sgn_c, jnp.float32)
    sin_ref[:, :] = lax.bitcast_convert_type(
        lax.bitcast_convert_type(s0, jnp.int32) ^ sgn_s, jnp.float32)


def kernel(position_ids, inv_freq):
    b, s = position_ids.shape
    total = b * s
    pos = position_ids.reshape(total, 1)
    fq = (inv_freq * (2.0 / jnp.pi)).astype(jnp.float32)
    freq = jnp.concatenate([fq, fq]).reshape(1, _DIM)
    grid = (total // _BLK,)
    cos, sin = pl.pallas_call(
        _rope_kernel,
        grid=grid,
        in_specs=[
            pl.BlockSpec((_BLK, 1), lambda i: (i, 0)),
            pl.BlockSpec((1, _DIM), lambda i: (0, 0)),
        ],
        out_specs=[
            pl.BlockSpec((_BLK, _DIM), lambda i: (i, 0)),
            pl.BlockSpec((_BLK, _DIM), lambda i: (i, 0)),
        ],
        out_shape=[
            jax.ShapeDtypeStruct((total, _DIM), jnp.float32),
            jax.ShapeDtypeStruct((total, _DIM), jnp.float32),
        ],
        compiler_params=pltpu.CompilerParams(
            dimension_semantics=("parallel",),
        ),
    )(pos, freq)
    return (cos.reshape(b, s, 1, _DIM), sin.reshape(b, s, 1, _DIM))


# speedup vs baseline: 4.1468x; 1.1073x over previous
import jax
import jax.numpy as jnp
from jax import lax
from jax.experimental import pallas as pl
from jax.experimental.pallas import tpu as pltpu

_BLK = 4096
_DIM = 128

_C1 = -1.2337005501361697
_C2 = 0.25366950790104696
_C3 = -0.020863480763352957
_S0 = 1.5707963267948966
_S1 = -0.6459640975062462
_S2 = 0.07969262624616703
_S3 = -0.0046817541353186846


def _rope_kernel(pos_ref, freq_ref, cos_ref, sin_ref):
    pos = pos_ref[:, :].astype(jnp.float32)
    x = pos * freq_ref[0:1, :]
    k = jnp.round(x)
    t = x - k
    q = k.astype(jnp.int32)
    t2 = t * t
    cp = 1.0 + t2 * (_C1 + t2 * (_C2 + t2 * _C3))
    sp = t * (_S0 + t2 * (_S1 + t2 * (_S2 + t2 * _S3)))
    swap = (q & 1) != 0
    c0 = jnp.where(swap, sp, cp)
    s0 = jnp.where(swap, cp, sp)
    sgn_c = ((q + 1) & 2) << 30
    sgn_s = (q & 2) << 30
    cos_ref[:, :] = lax.bitcast_convert_type(
        lax.bitcast_convert_type(c0, jnp.int32) ^ sgn_c, jnp.float32)
    sin_ref[:, :] = lax.bitcast_convert_type(
        lax.bitcast_convert_type(s0, jnp.int32) ^ sgn_s, jnp.float32)


def kernel(position_ids, inv_freq):
    b, s = position_ids.shape
    total = b * s
    pos = position_ids.reshape(total, 1)
    fq = (inv_freq * (2.0 / jnp.pi)).astype(jnp.float32)
    freq = jnp.concatenate([fq, fq]).reshape(1, _DIM)
    grid = (total // _BLK,)
    cos, sin = pl.pallas_call(
        _rope_kernel,
        grid=grid,
        in_specs=[
            pl.BlockSpec((_BLK, 1), lambda i: (i, 0)),
            pl.BlockSpec((1, _DIM), lambda i: (0, 0)),
        ],
        out_specs=[
            pl.BlockSpec((_BLK, _DIM), lambda i: (i, 0)),
            pl.BlockSpec((_BLK, _DIM), lambda i: (i, 0)),
        ],
        out_shape=[
            jax.ShapeDtypeStruct((total, _DIM), jnp.float32),
            jax.ShapeDtypeStruct((total, _DIM), jnp.float32),
        ],
        compiler_params=pltpu.CompilerParams(
            dimension_semantics=("parallel",),
        ),
    )(pos, freq)
    return (cos.reshape(b, s, 1, _DIM), sin.reshape(b, s, 1, _DIM))


# speedup vs baseline: 4.2398x; 1.0224x over previous
import jax
import jax.numpy as jnp
from jax import lax
from jax.experimental import pallas as pl
from jax.experimental.pallas import tpu as pltpu

_BLK = 8192
_DIM = 128

_C1 = -1.2337005501361697
_C2 = 0.25366950790104696
_C3 = -0.020863480763352957
_S0 = 1.5707963267948966
_S1 = -0.6459640975062462
_S2 = 0.07969262624616703
_S3 = -0.0046817541353186846


def _rope_kernel(pos_ref, freq_ref, cos_ref, sin_ref):
    pos = pos_ref[:, :].astype(jnp.float32)
    x = pos * freq_ref[0:1, :]
    k = jnp.round(x)
    t = x - k
    q = k.astype(jnp.int32)
    t2 = t * t
    cp = 1.0 + t2 * (_C1 + t2 * (_C2 + t2 * _C3))
    sp = t * (_S0 + t2 * (_S1 + t2 * (_S2 + t2 * _S3)))
    swap = (q & 1) != 0
    c0 = jnp.where(swap, sp, cp)
    s0 = jnp.where(swap, cp, sp)
    sgn_c = ((q + 1) & 2) << 30
    sgn_s = (q & 2) << 30
    cos_ref[:, :] = lax.bitcast_convert_type(
        lax.bitcast_convert_type(c0, jnp.int32) ^ sgn_c, jnp.float32)
    sin_ref[:, :] = lax.bitcast_convert_type(
        lax.bitcast_convert_type(s0, jnp.int32) ^ sgn_s, jnp.float32)


def kernel(position_ids, inv_freq):
    b, s = position_ids.shape
    total = b * s
    pos = position_ids.reshape(total, 1)
    fq = (inv_freq * (2.0 / jnp.pi)).astype(jnp.float32)
    freq = jnp.concatenate([fq, fq]).reshape(1, _DIM)
    grid = (total // _BLK,)
    cos, sin = pl.pallas_call(
        _rope_kernel,
        grid=grid,
        in_specs=[
            pl.BlockSpec((_BLK, 1), lambda i: (i, 0)),
            pl.BlockSpec((1, _DIM), lambda i: (0, 0)),
        ],
        out_specs=[
            pl.BlockSpec((_BLK, _DIM), lambda i: (i, 0)),
            pl.BlockSpec((_BLK, _DIM), lambda i: (i, 0)),
        ],
        out_shape=[
            jax.ShapeDtypeStruct((total, _DIM), jnp.float32),
            jax.ShapeDtypeStruct((total, _DIM), jnp.float32),
        ],
        compiler_params=pltpu.CompilerParams(
            dimension_semantics=("parallel",),
        ),
    )(pos, freq)
    return (cos.reshape(b, s, 1, _DIM), sin.reshape(b, s, 1, _DIM))
